# baseline (speedup 1.0000x reference)
import functools

import jax
import jax.numpy as jnp
from jax import lax
from jax.experimental import pallas as pl
from jax.experimental.pallas import tpu as pltpu
from jax.experimental.pallas import tpu_sc as plsc

_D = 32
_NC = 2
_NS = 16
_NW = _NC * _NS


@functools.lru_cache(maxsize=None)
def _make_gather(BSZ, HIST, D, RPC):
    B = BSZ * HIST
    C = RPC * HIST
    b_per_w = B // _NW
    n_chunks = b_per_w // C
    assert n_chunks * C == b_per_w and n_chunks % 2 == 0, (B, C)
    mesh = plsc.VectorSubcoreMesh(core_axis_name="c", subcore_axis_name="s")

    @functools.partial(
        pl.kernel,
        mesh=mesh,
        out_type=jax.ShapeDtypeStruct((BSZ, HIST, D), jnp.float32),
        scratch_types=[
            pltpu.VMEM((2, C), jnp.int32),
            pltpu.VMEM((2, C, D), jnp.float32),
            pltpu.SemaphoreType.DMA((2,)),
            pltpu.SemaphoreType.DMA((2,)),
            pltpu.SemaphoreType.DMA((2,)),
        ],
        compiler_params=pltpu.CompilerParams(use_tc_tiling_on_sc=False),
    )
    def k(table_hbm, idx_hbm, out_hbm, idx_v, rows_v, sem_i, sem_g, sem_s):
        wid = lax.axis_index("s") * _NC + lax.axis_index("c")
        base = wid * b_per_w

        def idx_cp(i, b):
            return pltpu.make_async_copy(
                idx_hbm.at[pl.ds(base + i * C, C)], idx_v.at[b], sem_i.at[b])

        def gather_cp(b):
            return pltpu.make_async_copy(
                table_hbm.at[idx_v.at[b]], rows_v.at[b], sem_g.at[b])

        def store_cps(i, b):
            r0 = (base + i * C) // HIST
            return [
                pltpu.make_async_copy(
                    rows_v.at[b].at[pl.ds(r * HIST, HIST)],
                    out_hbm.at[r0 + r], sem_s.at[b])
                for r in range(RPC)
            ]

        idx_cp(0, 0).start()
        idx_cp(1, 1).start()
        idx_cp(0, 0).wait()
        gather_cp(0).start()

        def chunk_step(i, b):
            gather_cp(b).wait()

            @pl.when(i + 2 < n_chunks)
            def _():
                idx_cp(i + 2, b).start()

            for cp in store_cps(i, b):
                cp.start()

            @pl.when(i + 1 < n_chunks)
            def _():
                idx_cp(i + 1, b ^ 1).wait()

                @pl.when(i >= 1)
                def _():
                    for cp in store_cps(i - 1, b ^ 1):
                        cp.wait()
                gather_cp(b ^ 1).start()

        def body(g, carry):
            chunk_step(2 * g, 0)
            chunk_step(2 * g + 1, 1)
            return carry

        lax.fori_loop(0, n_chunks // 2, body, 0)
        for cp in store_cps(n_chunks - 2, 0):
            cp.wait()
        for cp in store_cps(n_chunks - 1, 1):
            cp.wait()

    return k


def kernel(sequence, table):
    bsz, hist = sequence.shape
    B = bsz * hist
    idx = sequence.reshape(B).astype(jnp.int32)
    return _make_gather(bsz, hist, _D, 8)(table, idx)


# speedup vs baseline: 1.0008x; 1.0008x over previous
import functools

import jax
import jax.numpy as jnp
from jax import lax
from jax.experimental import pallas as pl
from jax.experimental.pallas import tpu as pltpu
from jax.experimental.pallas import tpu_sc as plsc

_D = 32
_NC = 2
_NS = 16
_NW = _NC * _NS
_NB = 4


@functools.lru_cache(maxsize=None)
def _make_gather(BSZ, HIST, D, RPC):
    B = BSZ * HIST
    C = RPC * HIST
    b_per_w = B // _NW
    n_chunks = b_per_w // C
    assert n_chunks * C == b_per_w and n_chunks % _NB == 0, (B, C)
    mesh = plsc.VectorSubcoreMesh(core_axis_name="c", subcore_axis_name="s")

    @functools.partial(
        pl.kernel,
        mesh=mesh,
        out_type=jax.ShapeDtypeStruct((BSZ, HIST, D), jnp.float32),
        scratch_types=[
            pltpu.VMEM((_NB, C), jnp.int32),
            pltpu.VMEM((_NB, C, D), jnp.float32),
            pltpu.SemaphoreType.DMA((_NB,)),
            pltpu.SemaphoreType.DMA((_NB,)),
            pltpu.SemaphoreType.DMA((_NB,)),
        ],
        compiler_params=pltpu.CompilerParams(use_tc_tiling_on_sc=False),
    )
    def k(table_hbm, idx_hbm, out_hbm, idx_v, rows_v, sem_i, sem_g, sem_s):
        wid = lax.axis_index("s") * _NC + lax.axis_index("c")
        base = wid * b_per_w

        def idx_cp(i, b):
            return pltpu.make_async_copy(
                idx_hbm.at[pl.ds(base + i * C, C)], idx_v.at[b], sem_i.at[b])

        def gather_cp(b):
            return pltpu.make_async_copy(
                table_hbm.at[idx_v.at[b]], rows_v.at[b], sem_g.at[b])

        def store_cps(i, b):
            r0 = (base + i * C) // HIST
            return [
                pltpu.make_async_copy(
                    rows_v.at[b].at[pl.ds(r * HIST, HIST)],
                    out_hbm.at[r0 + r], sem_s.at[b])
                for r in range(RPC)
            ]

        for b in range(_NB):
            idx_cp(b, b).start()
        idx_cp(0, 0).wait()
        gather_cp(0).start()
        idx_cp(1, 1).wait()
        gather_cp(1).start()

        def chunk_step(i, b):
            gather_cp(b).wait()

            @pl.when(i + _NB < n_chunks)
            def _():
                idx_cp(i + _NB, b).start()

            @pl.when(i + 2 < n_chunks)
            def _():
                idx_cp(i + 2, (b + 2) % _NB).wait()

                @pl.when(i >= 2)
                def _():
                    for cp in store_cps(i - 2, (b + 2) % _NB):
                        cp.wait()
                gather_cp((b + 2) % _NB).start()

            for cp in store_cps(i, b):
                cp.start()

        def body(g, carry):
            for u in range(_NB):
                chunk_step(_NB * g + u, u)
            return carry

        lax.fori_loop(0, n_chunks // _NB, body, 0)
        for i in range(n_chunks - 2, n_chunks):
            for cp in store_cps(i, i % _NB):
                cp.wait()

    return k


def kernel(sequence, table):
    bsz, hist = sequence.shape
    B = bsz * hist
    idx = sequence.reshape(B).astype(jnp.int32)
    return _make_gather(bsz, hist, _D, 4)(table, idx)
